# baseline (speedup 1.0000x reference)
import functools

import jax
import jax.numpy as jnp
from jax import lax
from jax.experimental import pallas as pl
from jax.experimental.pallas import tpu as pltpu
from jax.experimental.pallas import tpu_sc as plsc

N = 10000
D = 128
E = 320000
CH = 128
NC = 2
NS = 16
NW = NC * NS
NCH = 80
E_PAD = NW * CH * NCH
NPAD = 10112
RPT = NPAD // NS

_mesh = plsc.VectorSubcoreMesh(core_axis_name="c", subcore_axis_name="s")


@functools.partial(
    pl.kernel,
    out_type=jax.ShapeDtypeStruct((NC, NPAD, D), jnp.float32),
    mesh=_mesh,
    scratch_types=[
        pltpu.VMEM((NCH, CH), jnp.int32),
        pltpu.VMEM((CH, D), jnp.float32),
        pltpu.VMEM_SHARED((NPAD, D), jnp.float32),
        pltpu.SemaphoreType.DMA,
    ],
)
def _deg_kernel(dst_hbm, ones_hbm, zeros_hbm, out_hbm, didx, ones_v, acc, sem):
    c = lax.axis_index("c")
    s = lax.axis_index("s")
    wid = s * NC + c
    pltpu.sync_copy(zeros_hbm.at[pl.ds(s * RPT, RPT)], acc.at[pl.ds(s * RPT, RPT)])
    pltpu.sync_copy(ones_hbm, ones_v)
    pltpu.sync_copy(dst_hbm.at[pl.ds(wid * NCH, NCH)], didx)
    plsc.subcore_barrier()

    def body(i, carry):
        pltpu.sync_copy(ones_v, acc.at[didx.at[i]], add=True)
        return carry

    lax.fori_loop(0, NCH, body, 0)
    plsc.subcore_barrier()
    pltpu.sync_copy(acc.at[pl.ds(s * RPT, RPT)], out_hbm.at[c, pl.ds(s * RPT, RPT)])


@functools.partial(
    pl.kernel,
    out_type=jax.ShapeDtypeStruct((NC, NPAD, D), jnp.float32),
    mesh=_mesh,
    scratch_types=[
        pltpu.VMEM((CH,), jnp.int32),
        pltpu.VMEM((CH,), jnp.int32),
        pltpu.VMEM((CH, D), jnp.float32),
        pltpu.VMEM((CH, D), jnp.float32),
        pltpu.VMEM_SHARED((NPAD, D), jnp.float32),
        pltpu.SemaphoreType.DMA,
        pltpu.SemaphoreType.DMA,
    ],
)
def _msg_kernel(src_hbm, dst_hbm, h_hbm, zeros_hbm, out_hbm,
                sidx, didx, rows0, rows1, acc, sem0, sem1):
    c = lax.axis_index("c")
    s = lax.axis_index("s")
    pltpu.sync_copy(zeros_hbm.at[pl.ds(s * RPT, RPT)], acc.at[pl.ds(s * RPT, RPT)])
    plsc.subcore_barrier()

    def run_block(base, hch):
        def body(i, carry):
            pltpu.sync_copy(src_hbm.at[base + i], sidx)
            pltpu.sync_copy(dst_hbm.at[base + i], didx)
            pltpu.async_copy(h_hbm.at[sidx], rows0, sem0).wait()
            pltpu.sync_copy(rows0, acc.at[didx], add=True)
            return carry

        lax.fori_loop(0, hch, body, 0)

    wid = s * NC + c
    run_block(wid * NCH, NCH)

    plsc.subcore_barrier()
    pltpu.sync_copy(acc.at[pl.ds(s * RPT, RPT)], out_hbm.at[c, pl.ds(s * RPT, RPT)])


def _tca_body(parts_ref, x_ref, w1_ref, dis_ref, h1p_ref):
    deg = jnp.sum(parts_ref[0, :N, :] + parts_ref[1, :N, :], axis=1,
                  keepdims=True) * (1.0 / D) + 1.0
    dis = lax.rsqrt(deg)
    dis_ref[...] = dis
    h1p_ref[...] = dis * jnp.dot(x_ref[...], w1_ref[...],
                                 preferred_element_type=jnp.float32)


_tca = pl.pallas_call(
    _tca_body,
    out_shape=(jax.ShapeDtypeStruct((N, 1), jnp.float32),
               jax.ShapeDtypeStruct((N, D), jnp.float32)),
)


def _tcb_body(acc_ref, dis_ref, h1p_ref, b1_ref, w2_ref, h2p_ref):
    dis = dis_ref[...]
    z = dis * (acc_ref[0, :N, :] + acc_ref[1, :N, :] + h1p_ref[...]) + b1_ref[...]
    z = jnp.maximum(z, 0.0)
    h2p_ref[...] = dis * jnp.dot(z, w2_ref[...],
                                 preferred_element_type=jnp.float32)


_tcb = pl.pallas_call(
    _tcb_body,
    out_shape=jax.ShapeDtypeStruct((N, D), jnp.float32),
)


def _tcc_body(acc_ref, dis_ref, h2p_ref, b2_ref, out_ref):
    out_ref[...] = (dis_ref[...]
                    * (acc_ref[0, :N, :] + acc_ref[1, :N, :] + h2p_ref[...])
                    + b2_ref[...])


_tcc = pl.pallas_call(
    _tcc_body,
    out_shape=jax.ShapeDtypeStruct((N, D), jnp.float32),
)


def kernel(x, adj, W1, b1, W2, b2):
    src = adj[0].astype(jnp.int32)
    dst = adj[1].astype(jnp.int32)
    pad = E_PAD - E
    src_p = jnp.concatenate([src, jnp.zeros((pad,), jnp.int32)])
    dst_p = jnp.concatenate([dst, jnp.full((pad,), NPAD - 1, jnp.int32)])
    src2d = src_p.reshape(E_PAD // CH, CH)
    dst2d = dst_p.reshape(E_PAD // CH, CH)
    onesD = jnp.ones((CH, D), jnp.float32)
    zerosD = jnp.zeros((NPAD, D), jnp.float32)

    deg_parts = _deg_kernel(dst2d, onesD, zerosD)
    dis, h1p = _tca(deg_parts, x, W1)
    acc1 = _msg_kernel(src2d, dst2d, h1p, zerosD)
    h2p = _tcb(acc1, dis, h1p, b1.reshape(1, D), W2)
    acc2 = _msg_kernel(src2d, dst2d, h2p, zerosD)
    out = _tcc(acc2, dis, h2p, b2.reshape(1, D))
    return out


# speedup vs baseline: 1.0352x; 1.0352x over previous
import functools

import jax
import jax.numpy as jnp
from jax import lax
from jax.experimental import pallas as pl
from jax.experimental.pallas import tpu as pltpu
from jax.experimental.pallas import tpu_sc as plsc

N = 10000
D = 128
E = 320000
CH = 128
NC = 2
NS = 16
NW = NC * NS
NCH = 80
E_PAD = NW * CH * NCH
NPAD = 10112
RPT = NPAD // NS

_mesh = plsc.VectorSubcoreMesh(core_axis_name="c", subcore_axis_name="s")


@functools.partial(
    pl.kernel,
    out_type=jax.ShapeDtypeStruct((NC, NPAD, D), jnp.float32),
    mesh=_mesh,
    scratch_types=[
        pltpu.VMEM((NCH, CH), jnp.int32),
        pltpu.VMEM((CH, D), jnp.float32),
        pltpu.VMEM_SHARED((NPAD, D), jnp.float32),
        pltpu.SemaphoreType.DMA,
    ],
)
def _deg_kernel(dst_hbm, ones_hbm, zeros_hbm, out_hbm, didx, ones_v, acc, sem):
    c = lax.axis_index("c")
    s = lax.axis_index("s")
    wid = s * NC + c
    pltpu.sync_copy(zeros_hbm.at[pl.ds(s * RPT, RPT)], acc.at[pl.ds(s * RPT, RPT)])
    pltpu.sync_copy(ones_hbm, ones_v)
    pltpu.sync_copy(dst_hbm.at[pl.ds(wid * NCH, NCH)], didx)
    plsc.subcore_barrier()

    def body(i, carry):
        pltpu.sync_copy(ones_v, acc.at[didx.at[i]], add=True)
        return carry

    lax.fori_loop(0, NCH, body, 0)
    plsc.subcore_barrier()
    pltpu.sync_copy(acc.at[pl.ds(s * RPT, RPT)], out_hbm.at[c, pl.ds(s * RPT, RPT)])


@functools.partial(
    pl.kernel,
    out_type=jax.ShapeDtypeStruct((NC, NPAD, D), jnp.float32),
    mesh=_mesh,
    scratch_types=[
        pltpu.VMEM((CH,), jnp.int32),
        pltpu.VMEM((CH,), jnp.int32),
        pltpu.VMEM((CH, D), jnp.float32),
        pltpu.VMEM((CH, D), jnp.float32),
        pltpu.VMEM_SHARED((NPAD, D), jnp.float32),
        pltpu.SemaphoreType.DMA,
        pltpu.SemaphoreType.DMA,
    ],
)
def _msg_kernel(src_hbm, dst_hbm, h_hbm, zeros_hbm, out_hbm,
                sidx, didx, rows0, rows1, acc, sem0, sem1):
    c = lax.axis_index("c")
    s = lax.axis_index("s")
    pltpu.sync_copy(zeros_hbm.at[pl.ds(s * RPT, RPT)], acc.at[pl.ds(s * RPT, RPT)])
    plsc.subcore_barrier()

    def run_block(base, hch):
        def body(i, carry):
            off = (base + i) * CH
            pltpu.sync_copy(src_hbm.at[pl.ds(off, CH)], sidx)
            pltpu.sync_copy(dst_hbm.at[pl.ds(off, CH)], didx)
            pltpu.async_copy(h_hbm.at[sidx], rows0, sem0).wait()
            pltpu.sync_copy(rows0, acc.at[didx], add=True)
            return carry

        lax.fori_loop(0, hch, body, 0)

    wid = s * NC + c
    run_block(wid * NCH, NCH)

    plsc.subcore_barrier()
    pltpu.sync_copy(acc.at[pl.ds(s * RPT, RPT)], out_hbm.at[c, pl.ds(s * RPT, RPT)])


def _tca_body(parts_ref, x_ref, w1_ref, dis_ref, h1p_ref):
    deg = jnp.sum(parts_ref[0, :N, :] + parts_ref[1, :N, :], axis=1,
                  keepdims=True) * (1.0 / D) + 1.0
    dis = lax.rsqrt(deg)
    dis_ref[...] = dis
    h1p_ref[...] = dis * jnp.dot(x_ref[...], w1_ref[...],
                                 preferred_element_type=jnp.float32)


_tca = pl.pallas_call(
    _tca_body,
    out_shape=(jax.ShapeDtypeStruct((N, 1), jnp.float32),
               jax.ShapeDtypeStruct((N, D), jnp.float32)),
)


def _tcb_body(acc_ref, dis_ref, h1p_ref, b1_ref, w2_ref, h2p_ref):
    dis = dis_ref[...]
    z = dis * (acc_ref[0, :N, :] + acc_ref[1, :N, :] + h1p_ref[...]) + b1_ref[...]
    z = jnp.maximum(z, 0.0)
    h2p_ref[...] = dis * jnp.dot(z, w2_ref[...],
                                 preferred_element_type=jnp.float32)


_tcb = pl.pallas_call(
    _tcb_body,
    out_shape=jax.ShapeDtypeStruct((N, D), jnp.float32),
)


def _tcc_body(acc_ref, dis_ref, h2p_ref, b2_ref, out_ref):
    out_ref[...] = (dis_ref[...]
                    * (acc_ref[0, :N, :] + acc_ref[1, :N, :] + h2p_ref[...])
                    + b2_ref[...])


_tcc = pl.pallas_call(
    _tcc_body,
    out_shape=jax.ShapeDtypeStruct((N, D), jnp.float32),
)


def kernel(x, adj, W1, b1, W2, b2):
    src = adj[0].astype(jnp.int32)
    dst = adj[1].astype(jnp.int32)
    pad = E_PAD - E
    src_p = jnp.concatenate([src, jnp.zeros((pad,), jnp.int32)])
    dst_p = jnp.concatenate([dst, jnp.full((pad,), NPAD - 1, jnp.int32)])
    src2d = src_p.reshape(E_PAD // CH, CH)
    dst2d = dst_p.reshape(E_PAD // CH, CH)
    onesD = jnp.ones((CH, D), jnp.float32)
    zerosD = jnp.zeros((NPAD, D), jnp.float32)

    deg_parts = _deg_kernel(dst2d, onesD, zerosD)
    dis, h1p = _tca(deg_parts, x, W1)
    acc1 = _msg_kernel(src_p, dst_p, h1p, zerosD)
    h2p = _tcb(acc1, dis, h1p, b1.reshape(1, D), W2)
    acc2 = _msg_kernel(src_p, dst_p, h2p, zerosD)
    out = _tcc(acc2, dis, h2p, b2.reshape(1, D))
    return out


# speedup vs baseline: 1.4766x; 1.4264x over previous
import functools

import jax
import jax.numpy as jnp
from jax import lax
from jax.experimental import pallas as pl
from jax.experimental.pallas import tpu as pltpu
from jax.experimental.pallas import tpu_sc as plsc

N = 10000
D = 128
E = 320000
CH = 128
NC = 2
NS = 16
NW = NC * NS
NCH = 80
E_PAD = NW * CH * NCH
NPAD = 10112
RPT = NPAD // NS

_mesh = plsc.VectorSubcoreMesh(core_axis_name="c", subcore_axis_name="s")


@functools.partial(
    pl.kernel,
    out_type=jax.ShapeDtypeStruct((NC, NPAD, D), jnp.float32),
    mesh=_mesh,
    scratch_types=[
        pltpu.VMEM((NCH, CH), jnp.int32),
        pltpu.VMEM((CH, D), jnp.float32),
        pltpu.VMEM_SHARED((NPAD, D), jnp.float32),
        pltpu.SemaphoreType.DMA,
    ],
)
def _deg_kernel(dst_hbm, ones_hbm, zeros_hbm, out_hbm, didx, ones_v, acc, sem):
    c = lax.axis_index("c")
    s = lax.axis_index("s")
    wid = s * NC + c
    pltpu.sync_copy(zeros_hbm.at[pl.ds(s * RPT, RPT)], acc.at[pl.ds(s * RPT, RPT)])
    pltpu.sync_copy(ones_hbm, ones_v)
    pltpu.sync_copy(dst_hbm.at[pl.ds(wid * NCH, NCH)], didx)
    plsc.subcore_barrier()

    def body(i, carry):
        pltpu.sync_copy(ones_v, acc.at[didx.at[i]], add=True)
        return carry

    lax.fori_loop(0, NCH, body, 0)
    plsc.subcore_barrier()
    pltpu.sync_copy(acc.at[pl.ds(s * RPT, RPT)], out_hbm.at[c, pl.ds(s * RPT, RPT)])


@functools.partial(
    pl.kernel,
    out_type=jax.ShapeDtypeStruct((NC, NPAD, D), jnp.float32),
    mesh=_mesh,
    scratch_types=[
        pltpu.VMEM((48, CH), jnp.int32),
        pltpu.VMEM((48, CH), jnp.int32),
        pltpu.VMEM((CH, D), jnp.float32),
        pltpu.VMEM((CH, D), jnp.float32),
        pltpu.VMEM_SHARED((NPAD, D), jnp.float32),
        pltpu.SemaphoreType.DMA,
        pltpu.SemaphoreType.DMA,
    ],
)
def _msg_kernel(src_hbm, dst_hbm, h_hbm, zeros_hbm, out_hbm,
                sidx, didx, rows0, rows1, acc, sem0, sem1):
    c = lax.axis_index("c")
    s = lax.axis_index("s")
    pltpu.sync_copy(zeros_hbm.at[pl.ds(s * RPT, RPT)], acc.at[pl.ds(s * RPT, RPT)])
    plsc.subcore_barrier()

    def run_block(base, hch):
        pltpu.sync_copy(src_hbm.at[pl.ds(base, hch)], sidx.at[pl.ds(0, hch)])
        pltpu.sync_copy(dst_hbm.at[pl.ds(base, hch)], didx.at[pl.ds(0, hch)])
        pltpu.async_copy(h_hbm.at[sidx.at[0]], rows0, sem0)
        pltpu.async_copy(h_hbm.at[sidx.at[1]], rows1, sem1)

        def body(j, carry):
            p = 2 * j
            pltpu.make_async_copy(h_hbm.at[sidx.at[p]], rows0, sem0).wait()
            pltpu.sync_copy(rows0, acc.at[didx.at[p]], add=True)
            pltpu.async_copy(h_hbm.at[sidx.at[p + 2]], rows0, sem0)
            pltpu.make_async_copy(h_hbm.at[sidx.at[p + 1]], rows1, sem1).wait()
            pltpu.sync_copy(rows1, acc.at[didx.at[p + 1]], add=True)
            pltpu.async_copy(h_hbm.at[sidx.at[p + 3]], rows1, sem1)
            return carry

        lax.fori_loop(0, hch // 2 - 1, body, 0)
        pltpu.make_async_copy(h_hbm.at[sidx.at[hch - 2]], rows0, sem0).wait()
        pltpu.sync_copy(rows0, acc.at[didx.at[hch - 2]], add=True)
        pltpu.make_async_copy(h_hbm.at[sidx.at[hch - 1]], rows1, sem1).wait()
        pltpu.sync_copy(rows1, acc.at[didx.at[hch - 1]], add=True)

    NCH_C0 = 144
    NCH_C1 = 160 - NCH_C0

    @pl.when(c == 0)
    def _():
        for blk in range(3):
            run_block(s * NCH_C0 + blk * 48, 48)

    @pl.when(c == 1)
    def _():
        for blk in range(2):
            run_block(NS * NCH_C0 + s * NCH_C1 + blk * 8, 8)

    plsc.subcore_barrier()
    pltpu.sync_copy(acc.at[pl.ds(s * RPT, RPT)], out_hbm.at[c, pl.ds(s * RPT, RPT)])


def _tca_body(parts_ref, x_ref, w1_ref, dis_ref, h1p_ref):
    deg = jnp.sum(parts_ref[0, :N, :] + parts_ref[1, :N, :], axis=1,
                  keepdims=True) * (1.0 / D) + 1.0
    dis = lax.rsqrt(deg)
    dis_ref[...] = dis
    h1p_ref[...] = dis * jnp.dot(x_ref[...], w1_ref[...],
                                 preferred_element_type=jnp.float32)


_tca = pl.pallas_call(
    _tca_body,
    out_shape=(jax.ShapeDtypeStruct((N, 1), jnp.float32),
               jax.ShapeDtypeStruct((N, D), jnp.float32)),
)


def _tcb_body(acc_ref, dis_ref, h1p_ref, b1_ref, w2_ref, h2p_ref):
    dis = dis_ref[...]
    z = dis * (acc_ref[0, :N, :] + acc_ref[1, :N, :] + h1p_ref[...]) + b1_ref[...]
    z = jnp.maximum(z, 0.0)
    h2p_ref[...] = dis * jnp.dot(z, w2_ref[...],
                                 preferred_element_type=jnp.float32)


_tcb = pl.pallas_call(
    _tcb_body,
    out_shape=jax.ShapeDtypeStruct((N, D), jnp.float32),
)


def _tcc_body(acc_ref, dis_ref, h2p_ref, b2_ref, out_ref):
    out_ref[...] = (dis_ref[...]
                    * (acc_ref[0, :N, :] + acc_ref[1, :N, :] + h2p_ref[...])
                    + b2_ref[...])


_tcc = pl.pallas_call(
    _tcc_body,
    out_shape=jax.ShapeDtypeStruct((N, D), jnp.float32),
)


def kernel(x, adj, W1, b1, W2, b2):
    src = adj[0].astype(jnp.int32)
    dst = adj[1].astype(jnp.int32)
    pad = E_PAD - E
    src_p = jnp.concatenate([src, jnp.zeros((pad,), jnp.int32)])
    dst_p = jnp.concatenate([dst, jnp.full((pad,), NPAD - 1, jnp.int32)])
    src2d = src_p.reshape(E_PAD // CH, CH)
    dst2d = dst_p.reshape(E_PAD // CH, CH)
    onesD = jnp.ones((CH, D), jnp.float32)
    zerosD = jnp.zeros((NPAD, D), jnp.float32)

    deg_parts = _deg_kernel(dst2d, onesD, zerosD)
    dis, h1p = _tca(deg_parts, x, W1)
    acc1 = _msg_kernel(src2d, dst2d, h1p, zerosD)
    h2p = _tcb(acc1, dis, h1p, b1.reshape(1, D), W2)
    acc2 = _msg_kernel(src2d, dst2d, h2p, zerosD)
    out = _tcc(acc2, dis, h2p, b2.reshape(1, D))
    return out


# speedup vs baseline: 1.4995x; 1.0155x over previous
import functools

import jax
import jax.numpy as jnp
from jax import lax
from jax.experimental import pallas as pl
from jax.experimental.pallas import tpu as pltpu
from jax.experimental.pallas import tpu_sc as plsc

N = 10000
D = 128
E = 320000
CH = 128
NC = 2
NS = 16
NW = NC * NS
NCH = 80
E_PAD = NW * CH * NCH
NPAD = 10112
RPT = NPAD // NS

_mesh = plsc.VectorSubcoreMesh(core_axis_name="c", subcore_axis_name="s")


@functools.partial(
    pl.kernel,
    out_type=jax.ShapeDtypeStruct((NC, NPAD, D), jnp.float32),
    mesh=_mesh,
    scratch_types=[
        pltpu.VMEM((NCH, CH), jnp.int32),
        pltpu.VMEM((CH, D), jnp.float32),
        pltpu.VMEM_SHARED((NPAD, D), jnp.float32),
        pltpu.SemaphoreType.DMA,
    ],
)
def _deg_kernel(dst_hbm, ones_hbm, zeros_hbm, out_hbm, didx, ones_v, acc, sem):
    c = lax.axis_index("c")
    s = lax.axis_index("s")
    wid = s * NC + c
    pltpu.sync_copy(zeros_hbm.at[pl.ds(s * RPT, RPT)], acc.at[pl.ds(s * RPT, RPT)])
    pltpu.sync_copy(ones_hbm, ones_v)
    pltpu.sync_copy(dst_hbm.at[pl.ds(wid * NCH, NCH)], didx)
    plsc.subcore_barrier()

    def body(i, carry):
        pltpu.sync_copy(ones_v, acc.at[didx.at[i]], add=True)
        return carry

    lax.fori_loop(0, NCH, body, 0)
    plsc.subcore_barrier()
    pltpu.sync_copy(acc.at[pl.ds(s * RPT, RPT)], out_hbm.at[c, pl.ds(s * RPT, RPT)])


@functools.partial(
    pl.kernel,
    out_type=jax.ShapeDtypeStruct((NC, NPAD, D), jnp.float32),
    mesh=_mesh,
    scratch_types=[
        pltpu.VMEM((48, CH), jnp.int32),
        pltpu.VMEM((48, CH), jnp.int32),
        pltpu.VMEM((CH, D), jnp.float32),
        pltpu.VMEM((CH, D), jnp.float32),
        pltpu.VMEM_SHARED((NPAD, D), jnp.float32),
        pltpu.SemaphoreType.DMA,
        pltpu.SemaphoreType.DMA,
    ],
)
def _msg_kernel(src_hbm, dst_hbm, h_hbm, zeros_hbm, out_hbm,
                sidx, didx, rows0, rows1, acc, sem0, sem1):
    c = lax.axis_index("c")
    s = lax.axis_index("s")
    pltpu.sync_copy(zeros_hbm.at[pl.ds(s * RPT, RPT)], acc.at[pl.ds(s * RPT, RPT)])
    plsc.subcore_barrier()

    def run_block(base, hch):
        pltpu.sync_copy(src_hbm.at[pl.ds(base, hch)], sidx.at[pl.ds(0, hch)])
        pltpu.sync_copy(dst_hbm.at[pl.ds(base, hch)], didx.at[pl.ds(0, hch)])
        pltpu.async_copy(h_hbm.at[sidx.at[0]], rows0, sem0)
        pltpu.async_copy(h_hbm.at[sidx.at[1]], rows1, sem1)

        def body(j, carry):
            p = 2 * j
            pltpu.make_async_copy(h_hbm.at[sidx.at[p]], rows0, sem0).wait()
            pltpu.sync_copy(rows0, acc.at[didx.at[p]], add=True)
            pltpu.async_copy(h_hbm.at[sidx.at[p + 2]], rows0, sem0)
            pltpu.make_async_copy(h_hbm.at[sidx.at[p + 1]], rows1, sem1).wait()
            pltpu.sync_copy(rows1, acc.at[didx.at[p + 1]], add=True)
            pltpu.async_copy(h_hbm.at[sidx.at[p + 3]], rows1, sem1)
            return carry

        lax.fori_loop(0, hch // 2 - 1, body, 0)
        pltpu.make_async_copy(h_hbm.at[sidx.at[hch - 2]], rows0, sem0).wait()
        pltpu.sync_copy(rows0, acc.at[didx.at[hch - 2]], add=True)
        pltpu.make_async_copy(h_hbm.at[sidx.at[hch - 1]], rows1, sem1).wait()
        pltpu.sync_copy(rows1, acc.at[didx.at[hch - 1]], add=True)

    NCH_C0 = 152
    NCH_C1 = 160 - NCH_C0

    @pl.when(c == 0)
    def _():
        for base, blk in ((0, 48), (48, 48), (96, 48), (144, 8)):
            run_block(s * NCH_C0 + base, blk)

    @pl.when(c == 1)
    def _():
        run_block(NS * NCH_C0 + s * NCH_C1, 8)

    plsc.subcore_barrier()
    pltpu.sync_copy(acc.at[pl.ds(s * RPT, RPT)], out_hbm.at[c, pl.ds(s * RPT, RPT)])


def _tca_body(parts_ref, x_ref, w1_ref, dis_ref, h1p_ref):
    deg = jnp.sum(parts_ref[0, :N, :] + parts_ref[1, :N, :], axis=1,
                  keepdims=True) * (1.0 / D) + 1.0
    dis = lax.rsqrt(deg)
    dis_ref[...] = dis
    h1p_ref[...] = dis * jnp.dot(x_ref[...], w1_ref[...],
                                 preferred_element_type=jnp.float32)


_tca = pl.pallas_call(
    _tca_body,
    out_shape=(jax.ShapeDtypeStruct((N, 1), jnp.float32),
               jax.ShapeDtypeStruct((N, D), jnp.float32)),
)


def _tcb_body(acc_ref, dis_ref, h1p_ref, b1_ref, w2_ref, h2p_ref):
    dis = dis_ref[...]
    z = dis * (acc_ref[0, :N, :] + acc_ref[1, :N, :] + h1p_ref[...]) + b1_ref[...]
    z = jnp.maximum(z, 0.0)
    h2p_ref[...] = dis * jnp.dot(z, w2_ref[...],
                                 preferred_element_type=jnp.float32)


_tcb = pl.pallas_call(
    _tcb_body,
    out_shape=jax.ShapeDtypeStruct((N, D), jnp.float32),
)


def _tcc_body(acc_ref, dis_ref, h2p_ref, b2_ref, out_ref):
    out_ref[...] = (dis_ref[...]
                    * (acc_ref[0, :N, :] + acc_ref[1, :N, :] + h2p_ref[...])
                    + b2_ref[...])


_tcc = pl.pallas_call(
    _tcc_body,
    out_shape=jax.ShapeDtypeStruct((N, D), jnp.float32),
)


def kernel(x, adj, W1, b1, W2, b2):
    src = adj[0].astype(jnp.int32)
    dst = adj[1].astype(jnp.int32)
    pad = E_PAD - E
    src_p = jnp.concatenate([src, jnp.zeros((pad,), jnp.int32)])
    dst_p = jnp.concatenate([dst, jnp.full((pad,), NPAD - 1, jnp.int32)])
    src2d = src_p.reshape(E_PAD // CH, CH)
    dst2d = dst_p.reshape(E_PAD // CH, CH)
    onesD = jnp.ones((CH, D), jnp.float32)
    zerosD = jnp.zeros((NPAD, D), jnp.float32)

    deg_parts = _deg_kernel(dst2d, onesD, zerosD)
    dis, h1p = _tca(deg_parts, x, W1)
    acc1 = _msg_kernel(src2d, dst2d, h1p, zerosD)
    h2p = _tcb(acc1, dis, h1p, b1.reshape(1, D), W2)
    acc2 = _msg_kernel(src2d, dst2d, h2p, zerosD)
    out = _tcc(acc2, dis, h2p, b2.reshape(1, D))
    return out
